# baseline (speedup 1.0000x reference)
import numpy as np
import jax
import jax.numpy as jnp
from jax.experimental import pallas as pl

_EDGES = [(11, 12), (11, 23), (12, 24), (23, 24), (23, 25), (25, 27),
          (24, 26), (26, 28), (11, 13), (13, 15), (12, 14), (14, 16)]
_J = 33
_D = 64
_GRP = 8
_ROWS_G = _J * _GRP
_G2 = 8
_R = _ROWS_G * _G2


def _build_m_block():
    nb = {i: [i] for i in range(_J)}
    for a, b in _EDGES:
        nb[a].append(b)
        nb[b].append(a)
    m = np.zeros((_J, _J), np.float32)
    for j, ks in nb.items():
        w = 1.0 / len(ks)
        for k in ks:
            m[j, k] = w
    return np.kron(np.eye(_GRP, dtype=np.float32), m)


_M_BLOCK = _build_m_block()


def _body(x_ref, v_ref, m_ref, wc_ref, bc_ref, whh_ref, bhh_ref, o_ref):
    x = x_ref[...]
    v = v_ref[...]
    wf = x * v
    m = m_ref[...]
    aggs = []
    for g in range(_G2):
        blk = wf[g * _ROWS_G:(g + 1) * _ROWS_G, :]
        aggs.append(jnp.dot(m, blk, preferred_element_type=jnp.float32))
    agg = jnp.concatenate(aggs, axis=0)
    gi = jnp.dot(agg, wc_ref[...],
                 preferred_element_type=jnp.float32) + bc_ref[...]
    gh = jnp.dot(x, whh_ref[...],
                 preferred_element_type=jnp.float32) + bhh_ref[...]
    rz = jax.nn.sigmoid(gi + gh)
    r = rz[:, :_D]
    z = rz[:, _D:2 * _D]
    n = jnp.tanh(gi[:, 2 * _D:] + r * gh[:, 2 * _D:])
    o_ref[...] = (1.0 - z) * n + z * x


def kernel(joint_feats, visibility, W_msg, b_msg, W_ih, W_hh, b_ih, b_hh):
    B, J, D = joint_feats.shape
    rows = B * J
    x2 = joint_feats.reshape(rows, D)
    v2 = visibility.reshape(rows, 1)
    Wc = W_ih @ W_msg
    bc = b_ih + W_ih @ b_msg
    pad = (-rows) % _R
    if pad:
        x2 = jnp.pad(x2, ((0, pad), (0, 0)))
        v2 = jnp.pad(v2, ((0, pad), (0, 0)))
    grid = (rows + pad) // _R
    out = pl.pallas_call(
        _body,
        grid=(grid,),
        in_specs=[
            pl.BlockSpec((_R, D), lambda i: (i, 0)),
            pl.BlockSpec((_R, 1), lambda i: (i, 0)),
            pl.BlockSpec((_ROWS_G, _ROWS_G), lambda i: (0, 0)),
            pl.BlockSpec((D, 3 * D), lambda i: (0, 0)),
            pl.BlockSpec((1, 3 * D), lambda i: (0, 0)),
            pl.BlockSpec((D, 3 * D), lambda i: (0, 0)),
            pl.BlockSpec((1, 3 * D), lambda i: (0, 0)),
        ],
        out_specs=pl.BlockSpec((_R, D), lambda i: (i, 0)),
        out_shape=jax.ShapeDtypeStruct((rows + pad, D), jnp.float32),
    )(x2, v2, jnp.asarray(_M_BLOCK), Wc.T, bc.reshape(1, 3 * D),
      W_hh.T, b_hh.reshape(1, 3 * D))
    return out[:rows].reshape(B, J, D)


# speedup vs baseline: 1.4987x; 1.4987x over previous
import numpy as np
import jax
import jax.numpy as jnp
from jax.experimental import pallas as pl

_EDGES = [(11, 12), (11, 23), (12, 24), (23, 24), (23, 25), (25, 27),
          (24, 26), (26, 28), (11, 13), (13, 15), (12, 14), (14, 16)]
_J = 33
_D = 64
_BB = 256
_NG = 4


def _graph_tables():
    nb = {i: [i] for i in range(_J)}
    for a, b in _EDGES:
        nb[a].append(b)
        nb[b].append(a)
    invdeg = np.array([1.0 / len(nb[j]) for j in range(_J)], np.float32)
    rolls = {}
    for j in range(_J - 1):
        for k in nb[j][1:]:
            dg, ds = j // 8, j % 8
            sg, ss = k // 8, k % 8
            a = (ss - ds) % 8
            rolls.setdefault((sg, a), []).append((dg, ds))
    return invdeg, rolls


_INVDEG, _ROLLS = _graph_tables()


def _const_table():
    rows = [_INVDEG[8 * g:8 * g + 8] for g in range(_NG)]
    index = {}
    for (sg, a), dsts in sorted(_ROLLS.items()):
        for dg in sorted({d for d, _ in dsts}):
            mval = np.zeros((8,), np.float32)
            for d, s in dsts:
                if d == dg:
                    mval[s] = _INVDEG[8 * dg + s]
            index[(sg, a, dg)] = len(rows)
            rows.append(mval)
    return np.stack(rows).reshape(len(rows), 8, 1), index


_CONSTS, _CIDX = _const_table()


def _body(x_ref, v_ref, c_ref, wc_ref, bc_ref, whh_ref, bhh_ref, o_ref):
    wc = wc_ref[...]
    bc = bc_ref[...]
    whh = whh_ref[...]
    bhh = bhh_ref[...]

    def gru(agg_flat, x_flat):
        gi = jnp.dot(agg_flat, wc, preferred_element_type=jnp.float32) + bc
        gh = jnp.dot(x_flat, whh, preferred_element_type=jnp.float32) + bhh
        rz = jax.nn.sigmoid(gi + gh)
        r = rz[:, :_D]
        z = rz[:, _D:2 * _D]
        n = jnp.tanh(gi[:, 2 * _D:] + r * gh[:, 2 * _D:])
        return (1.0 - z) * n + z * x_flat

    xs3 = []
    wfs3 = []
    for g in range(_NG):
        x3 = x_ref[:, 8 * g:8 * g + 8, :]
        v3 = v_ref[:, 8 * g:8 * g + 8][:, :, None]
        xs3.append(x3)
        wfs3.append(x3 * v3)

    aggs3 = []
    for g in range(_NG):
        aggs3.append(wfs3[g] * c_ref[g:g + 1, :, :])
    for (sg, a), dsts in sorted(_ROLLS.items()):
        rolled = jnp.roll(wfs3[sg], -a, axis=1)
        for dg in sorted({d for d, _ in dsts}):
            i = _CIDX[(sg, a, dg)]
            aggs3[dg] = aggs3[dg] + rolled * c_ref[i:i + 1, :, :]

    for g in range(_NG):
        x_flat = xs3[g].reshape(_BB * 8, _D)
        agg_flat = aggs3[g].reshape(_BB * 8, _D)
        out = gru(agg_flat, x_flat)
        o_ref[:, 8 * g:8 * g + 8, :] = out.reshape(_BB, 8, _D)

    x32 = x_ref[:, 32:33, :].reshape(_BB, _D)
    v32 = v_ref[:, 32:33]
    o_ref[:, 32:33, :] = gru(x32 * v32, x32).reshape(_BB, 1, _D)


def kernel(joint_feats, visibility, W_msg, b_msg, W_ih, W_hh, b_ih, b_hh):
    B, J, D = joint_feats.shape
    Wc = W_ih @ W_msg
    bc = b_ih + W_ih @ b_msg
    grid = B // _BB
    out = pl.pallas_call(
        _body,
        grid=(grid,),
        in_specs=[
            pl.BlockSpec((_BB, J, D), lambda i: (i, 0, 0)),
            pl.BlockSpec((_BB, J), lambda i: (i, 0)),
            pl.BlockSpec(_CONSTS.shape, lambda i: (0, 0, 0)),
            pl.BlockSpec((D, 3 * D), lambda i: (0, 0)),
            pl.BlockSpec((1, 3 * D), lambda i: (0, 0)),
            pl.BlockSpec((D, 3 * D), lambda i: (0, 0)),
            pl.BlockSpec((1, 3 * D), lambda i: (0, 0)),
        ],
        out_specs=pl.BlockSpec((_BB, J, D), lambda i: (i, 0, 0)),
        out_shape=jax.ShapeDtypeStruct((B, J, D), jnp.float32),
    )(joint_feats, visibility, jnp.asarray(_CONSTS), Wc.T,
      bc.reshape(1, 3 * D), W_hh.T, b_hh.reshape(1, 3 * D))
    return out


# speedup vs baseline: 1.5134x; 1.0098x over previous
import numpy as np
import jax
import jax.numpy as jnp
from jax.experimental import pallas as pl

_EDGES = [(11, 12), (11, 23), (12, 24), (23, 24), (23, 25), (25, 27),
          (24, 26), (26, 28), (11, 13), (13, 15), (12, 14), (14, 16)]
_J = 33
_D = 64
_BB = 512
_NG = 4


def _graph_tables():
    nb = {i: [i] for i in range(_J)}
    for a, b in _EDGES:
        nb[a].append(b)
        nb[b].append(a)
    invdeg = np.array([1.0 / len(nb[j]) for j in range(_J)], np.float32)
    rolls = {}
    for j in range(_J - 1):
        for k in nb[j][1:]:
            dg, ds = j // 8, j % 8
            sg, ss = k // 8, k % 8
            a = (ss - ds) % 8
            rolls.setdefault((sg, a), []).append((dg, ds))
    return invdeg, rolls


_INVDEG, _ROLLS = _graph_tables()


def _const_table():
    rows = [_INVDEG[8 * g:8 * g + 8] for g in range(_NG)]
    index = {}
    for (sg, a), dsts in sorted(_ROLLS.items()):
        for dg in sorted({d for d, _ in dsts}):
            mval = np.zeros((8,), np.float32)
            for d, s in dsts:
                if d == dg:
                    mval[s] = _INVDEG[8 * dg + s]
            index[(sg, a, dg)] = len(rows)
            rows.append(mval)
    return np.stack(rows).reshape(len(rows), 8, 1), index


_CONSTS, _CIDX = _const_table()


def _body(x_ref, v_ref, c_ref, wc_ref, bc_ref, whh_ref, bhh_ref, o_ref):
    wc = wc_ref[...]
    bc = bc_ref[...]
    whh = whh_ref[...]
    bhh = bhh_ref[...]

    def gru(agg_flat, x_flat):
        gi = jnp.dot(agg_flat, wc, preferred_element_type=jnp.float32) + bc
        gh = jnp.dot(x_flat, whh, preferred_element_type=jnp.float32) + bhh
        rz = jax.nn.sigmoid(gi + gh)
        r = rz[:, :_D]
        z = rz[:, _D:2 * _D]
        n = jnp.tanh(gi[:, 2 * _D:] + r * gh[:, 2 * _D:])
        return (1.0 - z) * n + z * x_flat

    xs3 = []
    wfs3 = []
    for g in range(_NG):
        x3 = x_ref[:, 8 * g:8 * g + 8, :]
        v3 = v_ref[:, 8 * g:8 * g + 8][:, :, None]
        xs3.append(x3)
        wfs3.append(x3 * v3)

    aggs3 = []
    for g in range(_NG):
        aggs3.append(wfs3[g] * c_ref[g:g + 1, :, :])
    for (sg, a), dsts in sorted(_ROLLS.items()):
        rolled = jnp.roll(wfs3[sg], -a, axis=1)
        for dg in sorted({d for d, _ in dsts}):
            i = _CIDX[(sg, a, dg)]
            aggs3[dg] = aggs3[dg] + rolled * c_ref[i:i + 1, :, :]

    for g in range(_NG):
        x_flat = xs3[g].reshape(_BB * 8, _D)
        agg_flat = aggs3[g].reshape(_BB * 8, _D)
        out = gru(agg_flat, x_flat)
        o_ref[:, 8 * g:8 * g + 8, :] = out.reshape(_BB, 8, _D)

    x32 = x_ref[:, 32:33, :].reshape(_BB, _D)
    v32 = v_ref[:, 32:33]
    o_ref[:, 32:33, :] = gru(x32 * v32, x32).reshape(_BB, 1, _D)


def kernel(joint_feats, visibility, W_msg, b_msg, W_ih, W_hh, b_ih, b_hh):
    B, J, D = joint_feats.shape
    Wc = W_ih @ W_msg
    bc = b_ih + W_ih @ b_msg
    grid = B // _BB
    out = pl.pallas_call(
        _body,
        grid=(grid,),
        in_specs=[
            pl.BlockSpec((_BB, J, D), lambda i: (i, 0, 0)),
            pl.BlockSpec((_BB, J), lambda i: (i, 0)),
            pl.BlockSpec(_CONSTS.shape, lambda i: (0, 0, 0)),
            pl.BlockSpec((D, 3 * D), lambda i: (0, 0)),
            pl.BlockSpec((1, 3 * D), lambda i: (0, 0)),
            pl.BlockSpec((D, 3 * D), lambda i: (0, 0)),
            pl.BlockSpec((1, 3 * D), lambda i: (0, 0)),
        ],
        out_specs=pl.BlockSpec((_BB, J, D), lambda i: (i, 0, 0)),
        out_shape=jax.ShapeDtypeStruct((B, J, D), jnp.float32),
    )(joint_feats, visibility, jnp.asarray(_CONSTS), Wc.T,
      bc.reshape(1, 3 * D), W_hh.T, b_hh.reshape(1, 3 * D))
    return out
